# Initial kernel scaffold; baseline (speedup 1.0000x reference)
import jax
import jax.numpy as jnp
from jax.experimental import pallas as pl


def kernel(atom_feat, atom_bond_feat, frag_feat, fbond_feat, atom_edge_index, atom_graph_ids, frag_edge_index, frag_graph_ids, eps, params):
    raise NotImplementedError("write your pallas kernel here")


# speedup vs baseline: 2.4522x; 2.4522x over previous
import functools

import jax
import jax.numpy as jnp
from jax import lax
from jax.experimental import pallas as pl
from jax.experimental.pallas import tpu as pltpu
from jax.experimental.pallas import tpu_sc as plsc

H_ATOM = 32
H_BOND = 32
H_FNODE = 64
LATENT = 1024
N_ATOMS = 10000
E_ATOM = 40000
N_FRAGS = 2000
E_FRAG = 4000
B_MOL = 64

NA_PAD = 10240
EA_PAD = 40960
NF_PAD = 2048
EF_PAD = 4096

NW = 32
NC = 2
NS = 16
W_CNT = 32

ACC_A = NA_PAD + 16
ACC_F = NF_PAD + 16
ACC_M = B_MOL + 16
CNT_OFF_ADST = 0
CNT_OFF_AGID = NA_PAD
CNT_OFF_FDST = NA_PAD + NF_PAD
CNT_OFF_FGID = NA_PAD + 2 * NF_PAD
CNT_TRASH = CNT_OFF_FGID + B_MOL
ACC_CNT = CNT_TRASH + 16


def _mesh():
    return plsc.VectorSubcoreMesh(core_axis_name="c", subcore_axis_name="s")


_SC_PARAMS = pltpu.CompilerParams(use_tc_tiling_on_sc=False)


def _pad_rows(x, rows):
    return jnp.pad(x, ((0, rows - x.shape[0]), (0, 0)))


def _pad_idx(idx, n, fill):
    return jnp.concatenate(
        [idx, jnp.full((n - idx.shape[0],), fill, jnp.int32)])


def _chunk_idx(idx, chunk):
    return idx.reshape(NW, -1, chunk)


def _sc_gather(table, idx3, width):
    nw, nch, c = idx3.shape
    rpw = nch * c
    out_rows = nw * rpw

    @functools.partial(
        pl.kernel, mesh=_mesh(), compiler_params=_SC_PARAMS,
        out_type=jax.ShapeDtypeStruct((out_rows, width), jnp.float32),
        scratch_types=[
            pltpu.VMEM((nch, c), jnp.int32),
            pltpu.VMEM((rpw, width), jnp.float32),
            pltpu.SemaphoreType.DMA,
        ])
    def k(table_hbm, idx_hbm, out_hbm, idx_v, rows_v, sem):
        wid = lax.axis_index("s") * NC + lax.axis_index("c")
        pltpu.sync_copy(idx_hbm.at[wid], idx_v)
        cps = [pltpu.async_copy(table_hbm.at[idx_v.at[j]],
                                rows_v.at[pl.ds(j * c, c)], sem)
               for j in range(nch)]
        for cp in cps:
            cp.wait()
        pltpu.sync_copy(rows_v, out_hbm.at[pl.ds(wid * rpw, rpw)])

    return k(table, idx3)


def _sc_scatter_add(data, idx3, acc_rows, width, zeros):
    nw, nch, c = idx3.shape
    rpw = nch * c
    rps = acc_rows // NS

    @functools.partial(
        pl.kernel, mesh=_mesh(), compiler_params=_SC_PARAMS,
        out_type=jax.ShapeDtypeStruct((NC, acc_rows, width), jnp.float32),
        scratch_types=[
            pltpu.VMEM((nch, c), jnp.int32),
            pltpu.VMEM((rpw, width), jnp.float32),
            pltpu.VMEM_SHARED((acc_rows, width), jnp.float32),
        ])
    def k(data_hbm, idx_hbm, zeros_hbm, out_hbm, idx_v, data_v, acc_s):
        cid = lax.axis_index("c")
        sid = lax.axis_index("s")
        wid = sid * NC + cid
        pltpu.sync_copy(zeros_hbm.at[pl.ds(sid * rps, rps)],
                        acc_s.at[pl.ds(sid * rps, rps)])
        pltpu.sync_copy(idx_hbm.at[wid], idx_v)
        pltpu.sync_copy(data_hbm.at[pl.ds(wid * rpw, rpw)], data_v)
        plsc.subcore_barrier()
        for j in range(nch):
            pltpu.sync_copy(data_v.at[pl.ds(j * c, c)],
                            acc_s.at[idx_v.at[j]], add=True)
        plsc.subcore_barrier()
        pltpu.sync_copy(acc_s.at[pl.ds(sid * rps, rps)],
                        out_hbm.at[cid, pl.ds(sid * rps, rps)])

    return k(data, idx3, zeros)


def _sc_count(idx3, acc_rows, zeros):
    nw, nch, c = idx3.shape
    rps = acc_rows // NS

    @functools.partial(
        pl.kernel, mesh=_mesh(), compiler_params=_SC_PARAMS,
        out_type=jax.ShapeDtypeStruct((NC, acc_rows, W_CNT), jnp.float32),
        scratch_types=[
            pltpu.VMEM((nch, c), jnp.int32),
            pltpu.VMEM((c, W_CNT), jnp.float32),
            pltpu.VMEM_SHARED((acc_rows, W_CNT), jnp.float32),
        ])
    def k(idx_hbm, zeros_hbm, out_hbm, idx_v, ones_v, acc_s):
        cid = lax.axis_index("c")
        sid = lax.axis_index("s")
        wid = sid * NC + cid
        pltpu.sync_copy(zeros_hbm.at[pl.ds(sid * rps, rps)],
                        acc_s.at[pl.ds(sid * rps, rps)])
        pltpu.sync_copy(idx_hbm.at[wid], idx_v)
        one = jnp.ones((16,), jnp.float32)
        for i in range(c):
            for w in range(W_CNT // 16):
                ones_v[i, pl.ds(w * 16, 16)] = one
        plsc.subcore_barrier()
        for j in range(nch):
            pltpu.sync_copy(ones_v, acc_s.at[idx_v.at[j]], add=True)
        plsc.subcore_barrier()
        pltpu.sync_copy(acc_s.at[pl.ds(sid * rps, rps)],
                        out_hbm.at[cid, pl.ds(sid * rps, rps)])

    return k(idx3, zeros)


def _tc_embed(x, wt, b, blk):
    rows, f = x.shape
    h = wt.shape[1]

    def body(x_ref, w_ref, b_ref, o_ref):
        o_ref[...] = jnp.dot(x_ref[...], w_ref[...],
                             preferred_element_type=jnp.float32) + b_ref[...]

    return pl.pallas_call(
        body,
        grid=(rows // blk,),
        in_specs=[pl.BlockSpec((blk, f), lambda i: (i, 0)),
                  pl.BlockSpec((f, h), lambda i: (0, 0)),
                  pl.BlockSpec((1, h), lambda i: (0, 0))],
        out_specs=pl.BlockSpec((blk, h), lambda i: (i, 0)),
        out_shape=jax.ShapeDtypeStruct((rows, h), jnp.float32),
    )(x, wt, b)


def _tc_edge_r(x, wt, b, w1t, b1, blk):
    rows, f = x.shape
    h = wt.shape[1]
    k = w1t.shape[1]

    def body(x_ref, w_ref, b_ref, w1_ref, b1_ref, o_ref):
        e = jnp.dot(x_ref[...], w_ref[...],
                    preferred_element_type=jnp.float32) + b_ref[...]
        o_ref[...] = jax.nn.relu(
            jnp.dot(e, w1_ref[...], preferred_element_type=jnp.float32)
            + b1_ref[...])

    return pl.pallas_call(
        body,
        grid=(rows // blk,),
        in_specs=[pl.BlockSpec((blk, f), lambda i: (i, 0)),
                  pl.BlockSpec((f, h), lambda i: (0, 0)),
                  pl.BlockSpec((1, h), lambda i: (0, 0)),
                  pl.BlockSpec((h, k), lambda i: (0, 0)),
                  pl.BlockSpec((1, k), lambda i: (0, 0))],
        out_specs=pl.BlockSpec((blk, k), lambda i: (i, 0)),
        out_shape=jax.ShapeDtypeStruct((rows, k), jnp.float32),
    )(x, wt, b, w1t, b1)


def _tc_msg(hs, r, mcat, hdim, kdim, blk):
    rows = hs.shape[0]

    def body(hs_ref, r_ref, m_ref, o_ref):
        hst = hs_ref[...].T
        rt = r_ref[...].T
        p = (hst[:, None, :] * rt[None, :, :]).reshape(hdim * kdim, blk)
        full = jnp.concatenate([p, hst], axis=0)
        ot = jnp.dot(m_ref[...], full, preferred_element_type=jnp.float32)
        o_ref[...] = ot.T

    return pl.pallas_call(
        body,
        grid=(rows // blk,),
        in_specs=[pl.BlockSpec((blk, hdim), lambda i: (i, 0)),
                  pl.BlockSpec((blk, kdim), lambda i: (i, 0)),
                  pl.BlockSpec((hdim, hdim * kdim + hdim), lambda i: (0, 0))],
        out_specs=pl.BlockSpec((blk, hdim), lambda i: (i, 0)),
        out_shape=jax.ShapeDtypeStruct((rows, hdim), jnp.float32),
    )(hs, r, mcat)


def _tc_gru(s2, cnt2, h, wih_t, whh_t, bih, bhh, hdim, blk):
    rows = h.shape[0]
    h3 = 3 * hdim

    def body(s_ref, c_ref, h_ref, wi_ref, wh_ref, bi_ref, bh_ref, o_ref):
        c = c_ref[0] + c_ref[1]
        icnt = 1.0 / jnp.maximum(c, 1.0)
        s = s_ref[0] + s_ref[1]
        if hdim == W_CNT:
            m = jax.nn.relu(s * icnt)
        else:
            m = jax.nn.relu(s * icnt[:, 0:1])
        hv = h_ref[...]
        gi = jnp.dot(m, wi_ref[...],
                     preferred_element_type=jnp.float32) + bi_ref[...]
        gh = jnp.dot(hv, wh_ref[...],
                     preferred_element_type=jnp.float32) + bh_ref[...]
        rg = jax.nn.sigmoid(gi[:, :hdim] + gh[:, :hdim])
        zg = jax.nn.sigmoid(gi[:, hdim:2 * hdim] + gh[:, hdim:2 * hdim])
        ng = jnp.tanh(gi[:, 2 * hdim:] + rg * gh[:, 2 * hdim:])
        o_ref[...] = (1.0 - zg) * ng + zg * hv

    return pl.pallas_call(
        body,
        grid=(rows // blk,),
        in_specs=[pl.BlockSpec((NC, blk, hdim), lambda i: (0, i, 0)),
                  pl.BlockSpec((NC, blk, W_CNT), lambda i: (0, i, 0)),
                  pl.BlockSpec((blk, hdim), lambda i: (i, 0)),
                  pl.BlockSpec((hdim, h3), lambda i: (0, 0)),
                  pl.BlockSpec((hdim, h3), lambda i: (0, 0)),
                  pl.BlockSpec((1, h3), lambda i: (0, 0)),
                  pl.BlockSpec((1, h3), lambda i: (0, 0))],
        out_specs=pl.BlockSpec((blk, hdim), lambda i: (i, 0)),
        out_shape=jax.ShapeDtypeStruct((rows, hdim), jnp.float32),
    )(s2, cnt2, h, wih_t, whh_t, bih, bhh)


def _tc_frag_init(x, wt, b, sm2, cnt2, blk):
    rows, f = x.shape
    h = wt.shape[1]

    def body(x_ref, w_ref, b_ref, s_ref, c_ref, o_ref):
        emb = jnp.dot(x_ref[...], w_ref[...],
                      preferred_element_type=jnp.float32) + b_ref[...]
        c = c_ref[0] + c_ref[1]
        mean = (s_ref[0] + s_ref[1]) * (1.0 / jnp.maximum(c, 1.0))
        o_ref[...] = jnp.concatenate([emb, mean], axis=1)

    return pl.pallas_call(
        body,
        grid=(rows // blk,),
        in_specs=[pl.BlockSpec((blk, f), lambda i: (i, 0)),
                  pl.BlockSpec((f, h), lambda i: (0, 0)),
                  pl.BlockSpec((1, h), lambda i: (0, 0)),
                  pl.BlockSpec((NC, blk, h), lambda i: (0, i, 0)),
                  pl.BlockSpec((NC, blk, W_CNT), lambda i: (0, i, 0))],
        out_specs=pl.BlockSpec((blk, 2 * h), lambda i: (i, 0)),
        out_shape=jax.ShapeDtypeStruct((rows, 2 * h), jnp.float32),
    )(x, wt, b, sm2, cnt2)


def _tc_final(sm2, cnt2, enc_wt, enc_b, eps):
    def body(s_ref, c_ref, w_ref, b_ref, e_ref, o1_ref, o2_ref, o3_ref):
        c = c_ref[0] + c_ref[1]
        icnt = 1.0 / jnp.maximum(c, 1.0)
        hm = (s_ref[0] + s_ref[1]) * icnt[:, 0:1]
        x = jnp.dot(hm, w_ref[...],
                    preferred_element_type=jnp.float32) + b_ref[...]
        mu = x[:, :LATENT]
        lv = x[:, LATENT:]
        std = jnp.exp(0.5 * lv)
        o1_ref[...] = mu + e_ref[...] * std
        o2_ref[...] = mu
        o3_ref[...] = lv

    out = pl.pallas_call(
        body,
        out_shape=[jax.ShapeDtypeStruct((B_MOL, LATENT), jnp.float32)] * 3,
    )(sm2[:, :B_MOL, :], cnt2[:, :B_MOL, :], enc_wt, enc_b, eps)
    return out


def kernel(atom_feat, atom_bond_feat, frag_feat, fbond_feat, atom_edge_index,
           atom_graph_ids, frag_edge_index, frag_graph_ids, eps, params):
    p = params
    amp, fmp = p['amp'], p['fmp']

    wa_t = p['emb_atom_W'].T
    ba = p['emb_atom_b'].reshape(1, -1)
    wb_t = p['emb_bond_W'].T
    bb = p['emb_bond_b'].reshape(1, -1)
    e1a_t = amp['e1_W'].T
    b1a = amp['e1_b'].reshape(1, -1)
    mt_a = amp['e2_W'].reshape(H_ATOM, H_ATOM, H_BOND).transpose(1, 0, 2)
    mt_a = mt_a.reshape(H_ATOM, H_ATOM * H_BOND)
    bt_a = amp['e2_b'].reshape(H_ATOM, H_ATOM).T
    mcat_a = jnp.concatenate([mt_a, bt_a], axis=1)
    wih_a = amp['gru_Wih'].T
    whh_a = amp['gru_Whh'].T
    bih_a = amp['gru_bih'].reshape(1, -1)
    bhh_a = amp['gru_bhh'].reshape(1, -1)

    wf_t = p['emb_frag_W'].T
    bf = p['emb_frag_b'].reshape(1, -1)
    wfb_t = p['emb_fbond_W'].T
    bfb = p['emb_fbond_b'].reshape(1, -1)
    e1f_t = fmp['e1_W'].T
    b1f = fmp['e1_b'].reshape(1, -1)
    mt_f = fmp['e2_W'].reshape(H_FNODE, H_FNODE, H_BOND).transpose(1, 0, 2)
    mt_f = mt_f.reshape(H_FNODE, H_FNODE * H_BOND)
    bt_f = fmp['e2_b'].reshape(H_FNODE, H_FNODE).T
    mcat_f = jnp.concatenate([mt_f, bt_f], axis=1)
    wih_f = fmp['gru_Wih'].T
    whh_f = fmp['gru_Whh'].T
    bih_f = fmp['gru_bih'].reshape(1, -1)
    bhh_f = fmp['gru_bhh'].reshape(1, -1)
    enc_wt = p['enc_W'].T
    enc_b = p['enc_b'].reshape(1, -1)

    a_src, a_dst = atom_edge_index[0], atom_edge_index[1]
    f_src, f_dst = frag_edge_index[0], frag_edge_index[1]
    src3_a = _chunk_idx(_pad_idx(a_src, EA_PAD, 0), 128)
    dst3_a = _chunk_idx(_pad_idx(a_dst, EA_PAD, NA_PAD), 128)
    src3_f = _chunk_idx(_pad_idx(f_src, EF_PAD, 0), 128)
    dst3_f = _chunk_idx(_pad_idx(f_dst, EF_PAD, NF_PAD), 128)
    gid3_a = _chunk_idx(_pad_idx(atom_graph_ids, NA_PAD, NF_PAD), 64)
    gid3_f = _chunk_idx(_pad_idx(frag_graph_ids, NF_PAD, B_MOL), 64)
    cnt_idx = jnp.concatenate([
        _pad_idx(a_dst, EA_PAD, CNT_TRASH),
        _pad_idx(atom_graph_ids + CNT_OFF_AGID, NA_PAD, CNT_TRASH),
        _pad_idx(f_dst + CNT_OFF_FDST, EF_PAD, CNT_TRASH),
        _pad_idx(frag_graph_ids + CNT_OFF_FGID, NF_PAD, CNT_TRASH),
    ])
    cnt3 = _chunk_idx(cnt_idx, 128)

    z_cnt = jnp.zeros((ACC_CNT, W_CNT), jnp.float32)
    z_a = jnp.zeros((ACC_A, H_ATOM), jnp.float32)
    z_f32 = jnp.zeros((ACC_F, H_ATOM), jnp.float32)
    z_f64 = jnp.zeros((ACC_F, H_FNODE), jnp.float32)
    z_m = jnp.zeros((ACC_M, H_FNODE), jnp.float32)

    cnts = _sc_count(cnt3, ACC_CNT, z_cnt)
    cnt_adst = cnts[:, CNT_OFF_ADST:CNT_OFF_ADST + NA_PAD, :]
    cnt_agid = cnts[:, CNT_OFF_AGID:CNT_OFF_AGID + NF_PAD, :]
    cnt_fdst = cnts[:, CNT_OFF_FDST:CNT_OFF_FDST + NF_PAD, :]
    cnt_fgid = cnts[:, CNT_OFF_FGID:CNT_OFF_FGID + B_MOL + 16, :]

    af = _pad_rows(atom_feat, NA_PAD)
    abf = _pad_rows(atom_bond_feat, EA_PAD)
    h_a = _tc_embed(af, wa_t, ba, 1024)
    r_a = _tc_edge_r(abf, wb_t, bb, e1a_t, b1a, 2048)
    for _ in range(2):
        hs = _sc_gather(h_a, src3_a, H_ATOM)
        msg = _tc_msg(hs, r_a, mcat_a, H_ATOM, H_BOND, 512)
        s2 = _sc_scatter_add(msg, dst3_a, ACC_A, H_ATOM, z_a)
        h_a = _tc_gru(s2[:, :NA_PAD, :], cnt_adst, h_a,
                      wih_a, whh_a, bih_a, bhh_a, H_ATOM, 1024)

    sm_a = _sc_scatter_add(h_a, gid3_a, ACC_F, H_ATOM, z_f32)
    ff = _pad_rows(frag_feat, NF_PAD)
    h_f = _tc_frag_init(ff, wf_t, bf, sm_a[:, :NF_PAD, :], cnt_agid, 1024)

    fbf = _pad_rows(fbond_feat, EF_PAD)
    r_f = _tc_edge_r(fbf, wfb_t, bfb, e1f_t, b1f, 2048)
    for _ in range(2):
        hs = _sc_gather(h_f, src3_f, H_FNODE)
        msg = _tc_msg(hs, r_f, mcat_f, H_FNODE, H_BOND, 512)
        s2 = _sc_scatter_add(msg, dst3_f, ACC_F, H_FNODE, z_f64)
        h_f = _tc_gru(s2[:, :NF_PAD, :], cnt_fdst, h_f,
                      wih_f, whh_f, bih_f, bhh_f, H_FNODE, 1024)

    sm_f = _sc_scatter_add(h_f, gid3_f, ACC_M, H_FNODE, z_m)
    o1, mu, lv = _tc_final(sm_f, cnt_fgid, enc_wt, enc_b, eps)
    return (o1, mu, lv)


# speedup vs baseline: 2.4900x; 1.0154x over previous
import functools

import jax
import jax.numpy as jnp
from jax import lax
from jax.experimental import pallas as pl
from jax.experimental.pallas import tpu as pltpu
from jax.experimental.pallas import tpu_sc as plsc

H_ATOM = 32
H_BOND = 32
H_FNODE = 64
LATENT = 1024
N_ATOMS = 10000
E_ATOM = 40000
N_FRAGS = 2000
E_FRAG = 4000
B_MOL = 64

NA_PAD = 10240
EA_PAD = 40960
NF_PAD = 2048
EF_PAD = 4096

NW = 32
NC = 2
NS = 16
W_CNT = 16

ACC_A = NA_PAD + 16
ACC_F = NF_PAD + 16
ACC_M = B_MOL + 16
CNT_OFF_ADST = 0
CNT_OFF_AGID = NA_PAD
CNT_OFF_FDST = NA_PAD + NF_PAD
CNT_OFF_FGID = NA_PAD + 2 * NF_PAD
CNT_TRASH = CNT_OFF_FGID + B_MOL
ACC_CNT = CNT_TRASH + 16


def _mesh():
    return plsc.VectorSubcoreMesh(core_axis_name="c", subcore_axis_name="s")


_SC_PARAMS = pltpu.CompilerParams(use_tc_tiling_on_sc=False)


def _pad_rows(x, rows):
    return jnp.pad(x, ((0, rows - x.shape[0]), (0, 0)))


def _pad_idx(idx, n, fill):
    return jnp.concatenate(
        [idx, jnp.full((n - idx.shape[0],), fill, jnp.int32)])


def _chunk_idx(idx, chunk):
    return idx.reshape(NW, -1, chunk)


def _sc_gather(table, idx3, width):
    nw, nch, c = idx3.shape
    rpw = nch * c
    out_rows = nw * rpw

    @functools.partial(
        pl.kernel, mesh=_mesh(), compiler_params=_SC_PARAMS,
        out_type=jax.ShapeDtypeStruct((out_rows, width), jnp.float32),
        scratch_types=[
            pltpu.VMEM((nch, c), jnp.int32),
            pltpu.VMEM((rpw, width), jnp.float32),
            pltpu.SemaphoreType.DMA,
        ])
    def k(table_hbm, idx_hbm, out_hbm, idx_v, rows_v, sem):
        wid = lax.axis_index("s") * NC + lax.axis_index("c")
        pltpu.sync_copy(idx_hbm.at[wid], idx_v)
        cps = [pltpu.async_copy(table_hbm.at[idx_v.at[j]],
                                rows_v.at[pl.ds(j * c, c)], sem)
               for j in range(nch)]
        for cp in cps:
            cp.wait()
        pltpu.sync_copy(rows_v, out_hbm.at[pl.ds(wid * rpw, rpw)])

    return k(table, idx3)


def _sc_scatter_add(data, idx3, acc_rows, width, zeros):
    nw, nch, c = idx3.shape
    rpw = nch * c
    rps = acc_rows // NS

    @functools.partial(
        pl.kernel, mesh=_mesh(), compiler_params=_SC_PARAMS,
        out_type=jax.ShapeDtypeStruct((NC, acc_rows, width), jnp.float32),
        scratch_types=[
            pltpu.VMEM((nch, c), jnp.int32),
            pltpu.VMEM((rpw, width), jnp.float32),
            pltpu.VMEM_SHARED((acc_rows, width), jnp.float32),
        ])
    def k(data_hbm, idx_hbm, zeros_hbm, out_hbm, idx_v, data_v, acc_s):
        cid = lax.axis_index("c")
        sid = lax.axis_index("s")
        wid = sid * NC + cid
        pltpu.sync_copy(zeros_hbm.at[pl.ds(sid * rps, rps)],
                        acc_s.at[pl.ds(sid * rps, rps)])
        pltpu.sync_copy(idx_hbm.at[wid], idx_v)
        pltpu.sync_copy(data_hbm.at[pl.ds(wid * rpw, rpw)], data_v)
        plsc.subcore_barrier()
        for j in range(nch):
            pltpu.sync_copy(data_v.at[pl.ds(j * c, c)],
                            acc_s.at[idx_v.at[j]], add=True)
        plsc.subcore_barrier()
        pltpu.sync_copy(acc_s.at[pl.ds(sid * rps, rps)],
                        out_hbm.at[cid, pl.ds(sid * rps, rps)])

    return k(data, idx3, zeros)


def _sc_scatter_and_count(data, idx3, cidx3, acc_rows, width, zeros, zeros_c):
    nw, nch, c = idx3.shape
    rpw = nch * c
    rps = acc_rows // NS
    _, cnch, cc = cidx3.shape
    crps = ACC_CNT // NS

    @functools.partial(
        pl.kernel, mesh=_mesh(), compiler_params=_SC_PARAMS,
        out_type=[jax.ShapeDtypeStruct((NC, acc_rows, width), jnp.float32),
                  jax.ShapeDtypeStruct((NC, ACC_CNT, W_CNT), jnp.float32)],
        scratch_types=[
            pltpu.VMEM((nch, c), jnp.int32),
            pltpu.VMEM((cnch, cc), jnp.int32),
            pltpu.VMEM((rpw, width), jnp.float32),
            pltpu.VMEM((cc, W_CNT), jnp.float32),
            pltpu.VMEM_SHARED((acc_rows, width), jnp.float32),
            pltpu.VMEM_SHARED((ACC_CNT, W_CNT), jnp.float32),
        ])
    def k(data_hbm, idx_hbm, cidx_hbm, zeros_hbm, zeros_c_hbm,
          out_hbm, outc_hbm, idx_v, cidx_v, data_v, ones_v, acc_s, accc_s):
        cid = lax.axis_index("c")
        sid = lax.axis_index("s")
        wid = sid * NC + cid
        pltpu.sync_copy(zeros_hbm.at[pl.ds(sid * rps, rps)],
                        acc_s.at[pl.ds(sid * rps, rps)])
        pltpu.sync_copy(zeros_c_hbm.at[pl.ds(sid * crps, crps)],
                        accc_s.at[pl.ds(sid * crps, crps)])
        pltpu.sync_copy(idx_hbm.at[wid], idx_v)
        pltpu.sync_copy(cidx_hbm.at[wid], cidx_v)
        pltpu.sync_copy(data_hbm.at[pl.ds(wid * rpw, rpw)], data_v)
        one = jnp.ones((16,), jnp.float32)
        for i in range(cc):
            ones_v[i, pl.ds(0, 16)] = one
        plsc.subcore_barrier()
        for j in range(nch):
            pltpu.sync_copy(data_v.at[pl.ds(j * c, c)],
                            acc_s.at[idx_v.at[j]], add=True)
        for j in range(cnch):
            pltpu.sync_copy(ones_v, accc_s.at[cidx_v.at[j]], add=True)
        plsc.subcore_barrier()
        pltpu.sync_copy(acc_s.at[pl.ds(sid * rps, rps)],
                        out_hbm.at[cid, pl.ds(sid * rps, rps)])
        pltpu.sync_copy(accc_s.at[pl.ds(sid * crps, crps)],
                        outc_hbm.at[cid, pl.ds(sid * crps, crps)])

    return k(data, idx3, cidx3, zeros, zeros_c)


def _tc_embed(x, wt, b, blk):
    rows, f = x.shape
    h = wt.shape[1]

    def body(x_ref, w_ref, b_ref, o_ref):
        o_ref[...] = jnp.dot(x_ref[...], w_ref[...],
                             preferred_element_type=jnp.float32) + b_ref[...]

    return pl.pallas_call(
        body,
        grid=(rows // blk,),
        in_specs=[pl.BlockSpec((blk, f), lambda i: (i, 0)),
                  pl.BlockSpec((f, h), lambda i: (0, 0)),
                  pl.BlockSpec((1, h), lambda i: (0, 0))],
        out_specs=pl.BlockSpec((blk, h), lambda i: (i, 0)),
        out_shape=jax.ShapeDtypeStruct((rows, h), jnp.float32),
    )(x, wt, b)


def _tc_edge_r(x, wt, b, w1t, b1, blk):
    rows, f = x.shape
    h = wt.shape[1]
    k = w1t.shape[1]

    def body(x_ref, w_ref, b_ref, w1_ref, b1_ref, o_ref):
        e = jnp.dot(x_ref[...], w_ref[...],
                    preferred_element_type=jnp.float32) + b_ref[...]
        o_ref[...] = jax.nn.relu(
            jnp.dot(e, w1_ref[...], preferred_element_type=jnp.float32)
            + b1_ref[...])

    return pl.pallas_call(
        body,
        grid=(rows // blk,),
        in_specs=[pl.BlockSpec((blk, f), lambda i: (i, 0)),
                  pl.BlockSpec((f, h), lambda i: (0, 0)),
                  pl.BlockSpec((1, h), lambda i: (0, 0)),
                  pl.BlockSpec((h, k), lambda i: (0, 0)),
                  pl.BlockSpec((1, k), lambda i: (0, 0))],
        out_specs=pl.BlockSpec((blk, k), lambda i: (i, 0)),
        out_shape=jax.ShapeDtypeStruct((rows, k), jnp.float32),
    )(x, wt, b, w1t, b1)


def _tc_msg(hs, r, mcat, hdim, kdim, blk):
    rows = hs.shape[0]

    def body(hs_ref, r_ref, m_ref, o_ref):
        hst = hs_ref[...].T
        rt = r_ref[...].T
        p = (hst[:, None, :] * rt[None, :, :]).reshape(hdim * kdim, blk)
        full = jnp.concatenate([p, hst], axis=0)
        ot = jnp.dot(m_ref[...], full, preferred_element_type=jnp.float32)
        o_ref[...] = ot.T

    return pl.pallas_call(
        body,
        grid=(rows // blk,),
        in_specs=[pl.BlockSpec((blk, hdim), lambda i: (i, 0)),
                  pl.BlockSpec((blk, kdim), lambda i: (i, 0)),
                  pl.BlockSpec((hdim, hdim * kdim + hdim), lambda i: (0, 0))],
        out_specs=pl.BlockSpec((blk, hdim), lambda i: (i, 0)),
        out_shape=jax.ShapeDtypeStruct((rows, hdim), jnp.float32),
    )(hs, r, mcat)


def _tc_gru(s2, cnts, coff, h, wih_t, whh_t, bih, bhh, hdim, blk):
    rows = h.shape[0]
    h3 = 3 * hdim
    sacc = s2.shape[1]
    cacc = cnts.shape[1]

    def body(s_ref, c_ref, h_ref, wi_ref, wh_ref, bi_ref, bh_ref, o_ref):
        c = c_ref[0] + c_ref[1]
        icnt = 1.0 / jnp.maximum(c[:, 0:1], 1.0)
        s = s_ref[0] + s_ref[1]
        m = jax.nn.relu(s * icnt)
        hv = h_ref[...]
        gi = jnp.dot(m, wi_ref[...],
                     preferred_element_type=jnp.float32) + bi_ref[...]
        gh = jnp.dot(hv, wh_ref[...],
                     preferred_element_type=jnp.float32) + bh_ref[...]
        rg = jax.nn.sigmoid(gi[:, :hdim] + gh[:, :hdim])
        zg = jax.nn.sigmoid(gi[:, hdim:2 * hdim] + gh[:, hdim:2 * hdim])
        ng = jnp.tanh(gi[:, 2 * hdim:] + rg * gh[:, 2 * hdim:])
        o_ref[...] = (1.0 - zg) * ng + zg * hv

    return pl.pallas_call(
        body,
        grid=(rows // blk,),
        in_specs=[pl.BlockSpec((NC, blk, hdim), lambda i: (0, i, 0)),
                  pl.BlockSpec((NC, blk, W_CNT), lambda i: (0, coff + i, 0)),
                  pl.BlockSpec((blk, hdim), lambda i: (i, 0)),
                  pl.BlockSpec((hdim, h3), lambda i: (0, 0)),
                  pl.BlockSpec((hdim, h3), lambda i: (0, 0)),
                  pl.BlockSpec((1, h3), lambda i: (0, 0)),
                  pl.BlockSpec((1, h3), lambda i: (0, 0))],
        out_specs=pl.BlockSpec((blk, hdim), lambda i: (i, 0)),
        out_shape=jax.ShapeDtypeStruct((rows, hdim), jnp.float32),
    )(s2, cnts, h, wih_t, whh_t, bih, bhh)


def _tc_frag_init(x, wt, b, sm2, cnts, coff, blk):
    rows, f = x.shape
    h = wt.shape[1]

    def body(x_ref, w_ref, b_ref, s_ref, c_ref, o_ref):
        emb = jnp.dot(x_ref[...], w_ref[...],
                      preferred_element_type=jnp.float32) + b_ref[...]
        c = c_ref[0] + c_ref[1]
        icnt = 1.0 / jnp.maximum(c[:, 0:1], 1.0)
        mean = (s_ref[0] + s_ref[1]) * icnt
        o_ref[...] = jnp.concatenate([emb, mean], axis=1)

    return pl.pallas_call(
        body,
        grid=(rows // blk,),
        in_specs=[pl.BlockSpec((blk, f), lambda i: (i, 0)),
                  pl.BlockSpec((f, h), lambda i: (0, 0)),
                  pl.BlockSpec((1, h), lambda i: (0, 0)),
                  pl.BlockSpec((NC, blk, h), lambda i: (0, i, 0)),
                  pl.BlockSpec((NC, blk, W_CNT), lambda i: (0, coff + i, 0))],
        out_specs=pl.BlockSpec((blk, 2 * h), lambda i: (i, 0)),
        out_shape=jax.ShapeDtypeStruct((rows, 2 * h), jnp.float32),
    )(x, wt, b, sm2, cnts)


def _tc_final(sm2, cnts, enc_wt, enc_b, eps):
    cblk = CNT_OFF_FGID // B_MOL

    def body(s_ref, c_ref, w_ref, b_ref, e_ref, o1_ref, o2_ref, o3_ref):
        c = c_ref[0] + c_ref[1]
        icnt = 1.0 / jnp.maximum(c[:, 0:1], 1.0)
        hm = (s_ref[0] + s_ref[1]) * icnt
        x = jnp.dot(hm, w_ref[...],
                    preferred_element_type=jnp.float32) + b_ref[...]
        mu = x[:, :LATENT]
        lv = x[:, LATENT:]
        std = jnp.exp(0.5 * lv)
        o1_ref[...] = mu + e_ref[...] * std
        o2_ref[...] = mu
        o3_ref[...] = lv

    h = sm2.shape[2]
    d2 = enc_wt.shape[1]
    return pl.pallas_call(
        body,
        grid=(1,),
        in_specs=[pl.BlockSpec((NC, B_MOL, h), lambda i: (0, 0, 0)),
                  pl.BlockSpec((NC, B_MOL, W_CNT), lambda i: (0, cblk, 0)),
                  pl.BlockSpec((h, d2), lambda i: (0, 0)),
                  pl.BlockSpec((1, d2), lambda i: (0, 0)),
                  pl.BlockSpec((B_MOL, LATENT), lambda i: (0, 0))],
        out_specs=[pl.BlockSpec((B_MOL, LATENT), lambda i: (0, 0))] * 3,
        out_shape=[jax.ShapeDtypeStruct((B_MOL, LATENT), jnp.float32)] * 3,
    )(sm2, cnts, enc_wt, enc_b, eps)


def kernel(atom_feat, atom_bond_feat, frag_feat, fbond_feat, atom_edge_index,
           atom_graph_ids, frag_edge_index, frag_graph_ids, eps, params):
    p = params
    amp, fmp = p['amp'], p['fmp']

    wa_t = p['emb_atom_W'].T
    ba = p['emb_atom_b'].reshape(1, -1)
    wb_t = p['emb_bond_W'].T
    bb = p['emb_bond_b'].reshape(1, -1)
    e1a_t = amp['e1_W'].T
    b1a = amp['e1_b'].reshape(1, -1)
    mt_a = amp['e2_W'].reshape(H_ATOM, H_ATOM, H_BOND).transpose(1, 0, 2)
    mt_a = mt_a.reshape(H_ATOM, H_ATOM * H_BOND)
    bt_a = amp['e2_b'].reshape(H_ATOM, H_ATOM).T
    mcat_a = jnp.concatenate([mt_a, bt_a], axis=1)
    wih_a = amp['gru_Wih'].T
    whh_a = amp['gru_Whh'].T
    bih_a = amp['gru_bih'].reshape(1, -1)
    bhh_a = amp['gru_bhh'].reshape(1, -1)

    wf_t = p['emb_frag_W'].T
    bf = p['emb_frag_b'].reshape(1, -1)
    wfb_t = p['emb_fbond_W'].T
    bfb = p['emb_fbond_b'].reshape(1, -1)
    e1f_t = fmp['e1_W'].T
    b1f = fmp['e1_b'].reshape(1, -1)
    mt_f = fmp['e2_W'].reshape(H_FNODE, H_FNODE, H_BOND).transpose(1, 0, 2)
    mt_f = mt_f.reshape(H_FNODE, H_FNODE * H_BOND)
    bt_f = fmp['e2_b'].reshape(H_FNODE, H_FNODE).T
    mcat_f = jnp.concatenate([mt_f, bt_f], axis=1)
    wih_f = fmp['gru_Wih'].T
    whh_f = fmp['gru_Whh'].T
    bih_f = fmp['gru_bih'].reshape(1, -1)
    bhh_f = fmp['gru_bhh'].reshape(1, -1)
    enc_wt = p['enc_W'].T
    enc_b = p['enc_b'].reshape(1, -1)

    a_src, a_dst = atom_edge_index[0], atom_edge_index[1]
    f_src, f_dst = frag_edge_index[0], frag_edge_index[1]
    src3_a = _chunk_idx(_pad_idx(a_src, EA_PAD, 0), 128)
    dst3_a = _chunk_idx(_pad_idx(a_dst, EA_PAD, NA_PAD), 128)
    src3_f = _chunk_idx(_pad_idx(f_src, EF_PAD, 0), 128)
    dst3_f = _chunk_idx(_pad_idx(f_dst, EF_PAD, NF_PAD), 128)
    gid3_a = _chunk_idx(_pad_idx(atom_graph_ids, NA_PAD, NF_PAD), 64)
    gid3_f = _chunk_idx(_pad_idx(frag_graph_ids, NF_PAD, B_MOL), 64)
    cnt_idx = jnp.concatenate([
        _pad_idx(a_dst, EA_PAD, CNT_TRASH),
        _pad_idx(atom_graph_ids + CNT_OFF_AGID, NA_PAD, CNT_TRASH),
        _pad_idx(f_dst + CNT_OFF_FDST, EF_PAD, CNT_TRASH),
        _pad_idx(frag_graph_ids + CNT_OFF_FGID, NF_PAD, CNT_TRASH),
    ])
    cnt3 = _chunk_idx(cnt_idx, 128)

    z_cnt = jnp.zeros((ACC_CNT, W_CNT), jnp.float32)
    z_a = jnp.zeros((ACC_A, H_ATOM), jnp.float32)
    z_f32 = jnp.zeros((ACC_F, H_ATOM), jnp.float32)
    z_f64 = jnp.zeros((ACC_F, H_FNODE), jnp.float32)
    z_m = jnp.zeros((ACC_M, H_FNODE), jnp.float32)

    af = _pad_rows(atom_feat, NA_PAD)
    abf = _pad_rows(atom_bond_feat, EA_PAD)
    h_a = _tc_embed(af, wa_t, ba, 1024)
    r_a = _tc_edge_r(abf, wb_t, bb, e1a_t, b1a, 2048)

    hs = _sc_gather(h_a, src3_a, H_ATOM)
    msg = _tc_msg(hs, r_a, mcat_a, H_ATOM, H_BOND, 512)
    s2, cnts = _sc_scatter_and_count(msg, dst3_a, cnt3, ACC_A, H_ATOM,
                                     z_a, z_cnt)
    h_a = _tc_gru(s2, cnts, 0, h_a, wih_a, whh_a, bih_a, bhh_a, H_ATOM, 1024)

    hs = _sc_gather(h_a, src3_a, H_ATOM)
    msg = _tc_msg(hs, r_a, mcat_a, H_ATOM, H_BOND, 512)
    s2 = _sc_scatter_add(msg, dst3_a, ACC_A, H_ATOM, z_a)
    h_a = _tc_gru(s2, cnts, 0, h_a, wih_a, whh_a, bih_a, bhh_a, H_ATOM, 1024)

    sm_a = _sc_scatter_add(h_a, gid3_a, ACC_F, H_ATOM, z_f32)
    ff = _pad_rows(frag_feat, NF_PAD)
    h_f = _tc_frag_init(ff, wf_t, bf, sm_a, cnts, CNT_OFF_AGID // 1024, 1024)

    fbf = _pad_rows(fbond_feat, EF_PAD)
    r_f = _tc_edge_r(fbf, wfb_t, bfb, e1f_t, b1f, 2048)
    for _ in range(2):
        hs = _sc_gather(h_f, src3_f, H_FNODE)
        msg = _tc_msg(hs, r_f, mcat_f, H_FNODE, H_BOND, 512)
        s2 = _sc_scatter_add(msg, dst3_f, ACC_F, H_FNODE, z_f64)
        h_f = _tc_gru(s2, cnts, CNT_OFF_FDST // 1024, h_f,
                      wih_f, whh_f, bih_f, bhh_f, H_FNODE, 1024)

    sm_f = _sc_scatter_add(h_f, gid3_f, ACC_M, H_FNODE, z_m)
    o1, mu, lv = _tc_final(sm_f, cnts, enc_wt, enc_b, eps)
    return (o1, mu, lv)


# speedup vs baseline: 2.7976x; 1.1236x over previous
import functools

import jax
import jax.numpy as jnp
from jax import lax
from jax.experimental import pallas as pl
from jax.experimental.pallas import tpu as pltpu
from jax.experimental.pallas import tpu_sc as plsc

H_ATOM = 32
H_BOND = 32
H_FNODE = 64
LATENT = 1024
N_ATOMS = 10000
E_ATOM = 40000
N_FRAGS = 2000
E_FRAG = 4000
B_MOL = 64

NA_PAD = 10240
EA_PAD = 40960
NF_PAD = 2048
EF_PAD = 4096

NW = 32
NC = 2
NS = 16
W_CNT = 16

ACC_A = NA_PAD + 16
ACC_F = NF_PAD + 16
ACC_M = B_MOL + 16
CNT_OFF_ADST = 0
CNT_OFF_AGID = NA_PAD
CNT_OFF_FDST = NA_PAD + NF_PAD
CNT_OFF_FGID = NA_PAD + 2 * NF_PAD
CNT_TRASH = CNT_OFF_FGID + B_MOL
ACC_CNT = CNT_TRASH + 16


def _mesh():
    return plsc.VectorSubcoreMesh(core_axis_name="c", subcore_axis_name="s")


_SC_PARAMS = pltpu.CompilerParams(use_tc_tiling_on_sc=False)


def _pad_rows(x, rows):
    return jnp.pad(x, ((0, rows - x.shape[0]), (0, 0)))


def _pad_idx(idx, n, fill):
    return jnp.concatenate(
        [idx, jnp.full((n - idx.shape[0],), fill, jnp.int32)])


def _chunk_idx(idx):
    return idx.reshape(-1, 128)


def _sc_gather(table, idx2, width):
    c = 128
    nch = idx2.shape[0] // NW
    rpw = nch * c
    out_rows = NW * rpw

    @functools.partial(
        pl.kernel, mesh=_mesh(), compiler_params=_SC_PARAMS,
        out_type=jax.ShapeDtypeStruct((out_rows, width), jnp.float32),
        scratch_types=[
            pltpu.VMEM((nch, c), jnp.int32),
            pltpu.VMEM((rpw, width), jnp.float32),
            pltpu.SemaphoreType.DMA,
        ])
    def k(table_hbm, idx_hbm, out_hbm, idx_v, rows_v, sem):
        wid = lax.axis_index("s") * NC + lax.axis_index("c")
        pltpu.sync_copy(idx_hbm.at[pl.ds(wid * nch, nch)], idx_v)
        cps = [pltpu.async_copy(table_hbm.at[idx_v.at[j]],
                                rows_v.at[pl.ds(j * c, c)], sem)
               for j in range(nch)]
        for cp in cps:
            cp.wait()
        pltpu.sync_copy(rows_v, out_hbm.at[pl.ds(wid * rpw, rpw)])

    return k(table, idx2)


def _sc_scatter_add(data, idx2, acc_rows, width, zeros, nwork=NW):
    c = 128
    nch = idx2.shape[0] // nwork
    rpw = nch * c
    rps = acc_rows // NS

    @functools.partial(
        pl.kernel, mesh=_mesh(), compiler_params=_SC_PARAMS,
        out_type=jax.ShapeDtypeStruct((NC, acc_rows, width), jnp.float32),
        scratch_types=[
            pltpu.VMEM((nch, c), jnp.int32),
            pltpu.VMEM((rpw, width), jnp.float32),
            pltpu.VMEM_SHARED((acc_rows, width), jnp.float32),
        ])
    def k(data_hbm, idx_hbm, zeros_hbm, out_hbm, idx_v, data_v, acc_s):
        cid = lax.axis_index("c")
        sid = lax.axis_index("s")
        wid = sid * NC + cid
        pltpu.sync_copy(zeros_hbm.at[pl.ds(sid * rps, rps)],
                        acc_s.at[pl.ds(sid * rps, rps)])

        @pl.when(wid < nwork)
        def _():
            pltpu.sync_copy(idx_hbm.at[pl.ds(wid * nch, nch)], idx_v)
            pltpu.sync_copy(data_hbm.at[pl.ds(wid * rpw, rpw)], data_v)
        plsc.subcore_barrier()

        @pl.when(wid < nwork)
        def _():
            for j in range(nch):
                pltpu.sync_copy(data_v.at[pl.ds(j * c, c)],
                                acc_s.at[idx_v.at[j]], add=True)
        plsc.subcore_barrier()
        pltpu.sync_copy(acc_s.at[pl.ds(sid * rps, rps)],
                        out_hbm.at[cid, pl.ds(sid * rps, rps)])

    return k(data, idx2, zeros)


def _sc_scatter_and_count(data, idx2, cidx2, acc_rows, width, zeros, zeros_c):
    c = cc = 128
    nch = idx2.shape[0] // NW
    cnch = cidx2.shape[0] // NW
    rpw = nch * c
    rps = acc_rows // NS
    crps = ACC_CNT // NS

    @functools.partial(
        pl.kernel, mesh=_mesh(), compiler_params=_SC_PARAMS,
        out_type=[jax.ShapeDtypeStruct((NC, acc_rows, width), jnp.float32),
                  jax.ShapeDtypeStruct((NC, ACC_CNT, W_CNT), jnp.float32)],
        scratch_types=[
            pltpu.VMEM((nch, c), jnp.int32),
            pltpu.VMEM((cnch, cc), jnp.int32),
            pltpu.VMEM((rpw, width), jnp.float32),
            pltpu.VMEM((cc, W_CNT), jnp.float32),
            pltpu.VMEM_SHARED((acc_rows, width), jnp.float32),
            pltpu.VMEM_SHARED((ACC_CNT, W_CNT), jnp.float32),
        ])
    def k(data_hbm, idx_hbm, cidx_hbm, zeros_hbm, zeros_c_hbm,
          out_hbm, outc_hbm, idx_v, cidx_v, data_v, ones_v, acc_s, accc_s):
        cid = lax.axis_index("c")
        sid = lax.axis_index("s")
        wid = sid * NC + cid
        pltpu.sync_copy(zeros_hbm.at[pl.ds(sid * rps, rps)],
                        acc_s.at[pl.ds(sid * rps, rps)])
        pltpu.sync_copy(zeros_c_hbm.at[pl.ds(sid * crps, crps)],
                        accc_s.at[pl.ds(sid * crps, crps)])
        pltpu.sync_copy(idx_hbm.at[pl.ds(wid * nch, nch)], idx_v)
        pltpu.sync_copy(cidx_hbm.at[pl.ds(wid * cnch, cnch)], cidx_v)
        pltpu.sync_copy(data_hbm.at[pl.ds(wid * rpw, rpw)], data_v)
        one = jnp.ones((16,), jnp.float32)
        for i in range(cc):
            ones_v[i, pl.ds(0, 16)] = one
        plsc.subcore_barrier()
        for j in range(nch):
            pltpu.sync_copy(data_v.at[pl.ds(j * c, c)],
                            acc_s.at[idx_v.at[j]], add=True)
        for j in range(cnch):
            pltpu.sync_copy(ones_v, accc_s.at[cidx_v.at[j]], add=True)
        plsc.subcore_barrier()
        pltpu.sync_copy(acc_s.at[pl.ds(sid * rps, rps)],
                        out_hbm.at[cid, pl.ds(sid * rps, rps)])
        pltpu.sync_copy(accc_s.at[pl.ds(sid * crps, crps)],
                        outc_hbm.at[cid, pl.ds(sid * crps, crps)])

    return k(data, idx2, cidx2, zeros, zeros_c)


def _tc_embed(x, wt, b, blk):
    rows, f = x.shape
    h = wt.shape[1]

    def body(x_ref, w_ref, b_ref, o_ref):
        o_ref[...] = jnp.dot(x_ref[...], w_ref[...],
                             preferred_element_type=jnp.float32) + b_ref[...]

    return pl.pallas_call(
        body,
        grid=(rows // blk,),
        in_specs=[pl.BlockSpec((blk, f), lambda i: (i, 0)),
                  pl.BlockSpec((f, h), lambda i: (0, 0)),
                  pl.BlockSpec((1, h), lambda i: (0, 0))],
        out_specs=pl.BlockSpec((blk, h), lambda i: (i, 0)),
        out_shape=jax.ShapeDtypeStruct((rows, h), jnp.float32),
    )(x, wt, b)


def _tc_edge_r(x, wt, b, w1t, b1, blk):
    rows, f = x.shape
    h = wt.shape[1]
    k = w1t.shape[1]

    def body(x_ref, w_ref, b_ref, w1_ref, b1_ref, o_ref):
        e = jnp.dot(x_ref[...], w_ref[...],
                    preferred_element_type=jnp.float32) + b_ref[...]
        o_ref[...] = jax.nn.relu(
            jnp.dot(e, w1_ref[...], preferred_element_type=jnp.float32)
            + b1_ref[...])

    return pl.pallas_call(
        body,
        grid=(rows // blk,),
        in_specs=[pl.BlockSpec((blk, f), lambda i: (i, 0)),
                  pl.BlockSpec((f, h), lambda i: (0, 0)),
                  pl.BlockSpec((1, h), lambda i: (0, 0)),
                  pl.BlockSpec((h, k), lambda i: (0, 0)),
                  pl.BlockSpec((1, k), lambda i: (0, 0))],
        out_specs=pl.BlockSpec((blk, k), lambda i: (i, 0)),
        out_shape=jax.ShapeDtypeStruct((rows, k), jnp.float32),
    )(x, wt, b, w1t, b1)


def _tc_msg(hs, r, mcat, hdim, kdim, blk):
    rows = hs.shape[0]

    def body(hs_ref, r_ref, m_ref, o_ref):
        hst = hs_ref[...].astype(jnp.bfloat16).T
        rt = r_ref[...].astype(jnp.bfloat16).T
        p = (hst[:, None, :] * rt[None, :, :]).reshape(hdim * kdim, blk)
        full = jnp.concatenate([p, hst], axis=0)
        ot = jnp.dot(m_ref[...], full, preferred_element_type=jnp.float32)
        o_ref[...] = ot.T

    return pl.pallas_call(
        body,
        grid=(rows // blk,),
        in_specs=[pl.BlockSpec((blk, hdim), lambda i: (i, 0)),
                  pl.BlockSpec((blk, kdim), lambda i: (i, 0)),
                  pl.BlockSpec((hdim, hdim * kdim + hdim), lambda i: (0, 0))],
        out_specs=pl.BlockSpec((blk, hdim), lambda i: (i, 0)),
        out_shape=jax.ShapeDtypeStruct((rows, hdim), jnp.float32),
    )(hs, r, mcat)


def _tc_gru(s2, cnts, coff, h, wih_t, whh_t, bih, bhh, hdim, blk):
    rows = h.shape[0]
    h3 = 3 * hdim
    sacc = s2.shape[1]
    cacc = cnts.shape[1]

    def body(s_ref, c_ref, h_ref, wi_ref, wh_ref, bi_ref, bh_ref, o_ref):
        c = c_ref[0] + c_ref[1]
        icnt = 1.0 / jnp.maximum(c[:, 0:1], 1.0)
        s = s_ref[0] + s_ref[1]
        m = jax.nn.relu(s * icnt)
        hv = h_ref[...]
        gi = jnp.dot(m, wi_ref[...],
                     preferred_element_type=jnp.float32) + bi_ref[...]
        gh = jnp.dot(hv, wh_ref[...],
                     preferred_element_type=jnp.float32) + bh_ref[...]
        rg = jax.nn.sigmoid(gi[:, :hdim] + gh[:, :hdim])
        zg = jax.nn.sigmoid(gi[:, hdim:2 * hdim] + gh[:, hdim:2 * hdim])
        ng = jnp.tanh(gi[:, 2 * hdim:] + rg * gh[:, 2 * hdim:])
        o_ref[...] = (1.0 - zg) * ng + zg * hv

    return pl.pallas_call(
        body,
        grid=(rows // blk,),
        in_specs=[pl.BlockSpec((NC, blk, hdim), lambda i: (0, i, 0)),
                  pl.BlockSpec((NC, blk, W_CNT), lambda i: (0, coff + i, 0)),
                  pl.BlockSpec((blk, hdim), lambda i: (i, 0)),
                  pl.BlockSpec((hdim, h3), lambda i: (0, 0)),
                  pl.BlockSpec((hdim, h3), lambda i: (0, 0)),
                  pl.BlockSpec((1, h3), lambda i: (0, 0)),
                  pl.BlockSpec((1, h3), lambda i: (0, 0))],
        out_specs=pl.BlockSpec((blk, hdim), lambda i: (i, 0)),
        out_shape=jax.ShapeDtypeStruct((rows, hdim), jnp.float32),
    )(s2, cnts, h, wih_t, whh_t, bih, bhh)


def _tc_frag_init(x, wt, b, sm2, cnts, coff, blk):
    rows, f = x.shape
    h = wt.shape[1]

    def body(x_ref, w_ref, b_ref, s_ref, c_ref, o_ref):
        emb = jnp.dot(x_ref[...], w_ref[...],
                      preferred_element_type=jnp.float32) + b_ref[...]
        c = c_ref[0] + c_ref[1]
        icnt = 1.0 / jnp.maximum(c[:, 0:1], 1.0)
        mean = (s_ref[0] + s_ref[1]) * icnt
        o_ref[...] = jnp.concatenate([emb, mean], axis=1)

    return pl.pallas_call(
        body,
        grid=(rows // blk,),
        in_specs=[pl.BlockSpec((blk, f), lambda i: (i, 0)),
                  pl.BlockSpec((f, h), lambda i: (0, 0)),
                  pl.BlockSpec((1, h), lambda i: (0, 0)),
                  pl.BlockSpec((NC, blk, h), lambda i: (0, i, 0)),
                  pl.BlockSpec((NC, blk, W_CNT), lambda i: (0, coff + i, 0))],
        out_specs=pl.BlockSpec((blk, 2 * h), lambda i: (i, 0)),
        out_shape=jax.ShapeDtypeStruct((rows, 2 * h), jnp.float32),
    )(x, wt, b, sm2, cnts)


def _tc_final(sm2, cnts, enc_wt, enc_b, eps):
    cblk = CNT_OFF_FGID // B_MOL

    def body(s_ref, c_ref, w_ref, b_ref, e_ref, o1_ref, o2_ref, o3_ref):
        c = c_ref[0] + c_ref[1]
        icnt = 1.0 / jnp.maximum(c[:, 0:1], 1.0)
        hm = (s_ref[0] + s_ref[1]) * icnt
        x = jnp.dot(hm, w_ref[...],
                    preferred_element_type=jnp.float32) + b_ref[...]
        mu = x[:, :LATENT]
        lv = x[:, LATENT:]
        std = jnp.exp(0.5 * lv)
        o1_ref[...] = mu + e_ref[...] * std
        o2_ref[...] = mu
        o3_ref[...] = lv

    h = sm2.shape[2]
    d2 = enc_wt.shape[1]
    return pl.pallas_call(
        body,
        grid=(1,),
        in_specs=[pl.BlockSpec((NC, B_MOL, h), lambda i: (0, 0, 0)),
                  pl.BlockSpec((NC, B_MOL, W_CNT), lambda i: (0, cblk, 0)),
                  pl.BlockSpec((h, d2), lambda i: (0, 0)),
                  pl.BlockSpec((1, d2), lambda i: (0, 0)),
                  pl.BlockSpec((B_MOL, LATENT), lambda i: (0, 0))],
        out_specs=[pl.BlockSpec((B_MOL, LATENT), lambda i: (0, 0))] * 3,
        out_shape=[jax.ShapeDtypeStruct((B_MOL, LATENT), jnp.float32)] * 3,
    )(sm2, cnts, enc_wt, enc_b, eps)


def kernel(atom_feat, atom_bond_feat, frag_feat, fbond_feat, atom_edge_index,
           atom_graph_ids, frag_edge_index, frag_graph_ids, eps, params):
    p = params
    amp, fmp = p['amp'], p['fmp']

    wa_t = p['emb_atom_W'].T
    ba = p['emb_atom_b'].reshape(1, -1)
    wb_t = p['emb_bond_W'].T
    bb = p['emb_bond_b'].reshape(1, -1)
    e1a_t = amp['e1_W'].T
    b1a = amp['e1_b'].reshape(1, -1)
    mt_a = amp['e2_W'].reshape(H_ATOM, H_ATOM, H_BOND).transpose(1, 0, 2)
    mt_a = mt_a.reshape(H_ATOM, H_ATOM * H_BOND)
    bt_a = amp['e2_b'].reshape(H_ATOM, H_ATOM).T
    mcat_a = jnp.concatenate([mt_a, bt_a], axis=1).astype(jnp.bfloat16)
    wih_a = amp['gru_Wih'].T
    whh_a = amp['gru_Whh'].T
    bih_a = amp['gru_bih'].reshape(1, -1)
    bhh_a = amp['gru_bhh'].reshape(1, -1)

    wf_t = p['emb_frag_W'].T
    bf = p['emb_frag_b'].reshape(1, -1)
    wfb_t = p['emb_fbond_W'].T
    bfb = p['emb_fbond_b'].reshape(1, -1)
    e1f_t = fmp['e1_W'].T
    b1f = fmp['e1_b'].reshape(1, -1)
    mt_f = fmp['e2_W'].reshape(H_FNODE, H_FNODE, H_BOND).transpose(1, 0, 2)
    mt_f = mt_f.reshape(H_FNODE, H_FNODE * H_BOND)
    bt_f = fmp['e2_b'].reshape(H_FNODE, H_FNODE).T
    mcat_f = jnp.concatenate([mt_f, bt_f], axis=1).astype(jnp.bfloat16)
    wih_f = fmp['gru_Wih'].T
    whh_f = fmp['gru_Whh'].T
    bih_f = fmp['gru_bih'].reshape(1, -1)
    bhh_f = fmp['gru_bhh'].reshape(1, -1)
    enc_wt = p['enc_W'].T
    enc_b = p['enc_b'].reshape(1, -1)

    a_src, a_dst = atom_edge_index[0], atom_edge_index[1]
    f_src, f_dst = frag_edge_index[0], frag_edge_index[1]
    src3_a = _chunk_idx(_pad_idx(a_src, EA_PAD, 0))
    dst3_a = _chunk_idx(_pad_idx(a_dst, EA_PAD, NA_PAD))
    src3_f = _chunk_idx(_pad_idx(f_src, EF_PAD, 0))
    dst3_f = _chunk_idx(_pad_idx(f_dst, EF_PAD, NF_PAD))
    gid3_a = _chunk_idx(_pad_idx(atom_graph_ids, NA_PAD, NF_PAD))
    gid3_f = _chunk_idx(_pad_idx(frag_graph_ids, NF_PAD, B_MOL))
    cnt_idx = jnp.concatenate([
        _pad_idx(a_dst, EA_PAD, CNT_TRASH),
        _pad_idx(atom_graph_ids + CNT_OFF_AGID, NA_PAD, CNT_TRASH),
        _pad_idx(f_dst + CNT_OFF_FDST, EF_PAD, CNT_TRASH),
        _pad_idx(frag_graph_ids + CNT_OFF_FGID, NF_PAD, CNT_TRASH),
    ])
    cnt3 = _chunk_idx(cnt_idx)

    z_cnt = jnp.zeros((ACC_CNT, W_CNT), jnp.float32)
    z_a = jnp.zeros((ACC_A, H_ATOM), jnp.float32)
    z_f32 = jnp.zeros((ACC_F, H_ATOM), jnp.float32)
    z_f64 = jnp.zeros((ACC_F, H_FNODE), jnp.float32)
    z_m = jnp.zeros((ACC_M, H_FNODE), jnp.float32)

    af = _pad_rows(atom_feat, NA_PAD)
    abf = _pad_rows(atom_bond_feat, EA_PAD)
    h_a = _tc_embed(af, wa_t, ba, 1024)
    r_a = _tc_edge_r(abf, wb_t, bb, e1a_t, b1a, 2048)

    hs = _sc_gather(h_a, src3_a, H_ATOM)
    msg = _tc_msg(hs, r_a, mcat_a, H_ATOM, H_BOND, 1024)
    s2, cnts = _sc_scatter_and_count(msg, dst3_a, cnt3, ACC_A, H_ATOM,
                                     z_a, z_cnt)
    h_a = _tc_gru(s2, cnts, 0, h_a, wih_a, whh_a, bih_a, bhh_a, H_ATOM, 1024)

    hs = _sc_gather(h_a, src3_a, H_ATOM)
    msg = _tc_msg(hs, r_a, mcat_a, H_ATOM, H_BOND, 1024)
    s2 = _sc_scatter_add(msg, dst3_a, ACC_A, H_ATOM, z_a)
    h_a = _tc_gru(s2, cnts, 0, h_a, wih_a, whh_a, bih_a, bhh_a, H_ATOM, 1024)

    sm_a = _sc_scatter_add(h_a, gid3_a, ACC_F, H_ATOM, z_f32, nwork=16)
    ff = _pad_rows(frag_feat, NF_PAD)
    h_f = _tc_frag_init(ff, wf_t, bf, sm_a, cnts, CNT_OFF_AGID // 1024, 1024)

    fbf = _pad_rows(fbond_feat, EF_PAD)
    r_f = _tc_edge_r(fbf, wfb_t, bfb, e1f_t, b1f, 2048)
    for _ in range(2):
        hs = _sc_gather(h_f, src3_f, H_FNODE)
        msg = _tc_msg(hs, r_f, mcat_f, H_FNODE, H_BOND, 1024)
        s2 = _sc_scatter_add(msg, dst3_f, ACC_F, H_FNODE, z_f64)
        h_f = _tc_gru(s2, cnts, CNT_OFF_FDST // 1024, h_f,
                      wih_f, whh_f, bih_f, bhh_f, H_FNODE, 1024)

    sm_f = _sc_scatter_add(h_f, gid3_f, ACC_M, H_FNODE, z_m, nwork=16)
    o1, mu, lv = _tc_final(sm_f, cnts, enc_wt, enc_b, eps)
    return (o1, mu, lv)
